# baseline (speedup 1.0000x reference)
import functools

import jax
import jax.numpy as jnp
from jax import lax
from jax.experimental import pallas as pl
from jax.experimental.pallas import tpu as pltpu, tpu_sc as plsc

BATCH = 16384
EMBED = 32
_NC = 2
_NS = 16
_NW = _NC * _NS
_BPW = BATCH // _NW
_CH = 128
_NCH = _BPW // _CH
_GROUPS = _BPW // 16


def _body(crow_hbm, arow_hbm, ctab_hbm, atab_hbm, cbias_hbm, abias_hbm,
          out_hbm, cidx_v, aidx_v, crows_v, arows_v, cb_v, ab_v, out_v, sem):
    wid = lax.axis_index("s") * _NC + lax.axis_index("c")

    pltpu.sync_copy(crow_hbm.at[pl.ds(wid * _NCH, _NCH)], cidx_v)
    pltpu.sync_copy(arow_hbm.at[pl.ds(wid * _NCH, _NCH)], aidx_v)

    copies = []
    for k in range(_NCH):
        copies.append(pltpu.async_copy(
            ctab_hbm.at[cidx_v.at[k]], crows_v.at[pl.ds(k * _CH, _CH)], sem))
        copies.append(pltpu.async_copy(
            atab_hbm.at[aidx_v.at[k]], arows_v.at[pl.ds(k * _CH, _CH)], sem))
        copies.append(pltpu.async_copy(cbias_hbm.at[cidx_v.at[k]], cb_v.at[k], sem))
        copies.append(pltpu.async_copy(abias_hbm.at[aidx_v.at[k]], ab_v.at[k], sem))
    for c in copies:
        c.wait()

    iota = lax.broadcasted_iota(jnp.int32, (16,), 0)

    def group(g, _):
        chunk = g // (_CH // 16)
        off = (g % (_CH // 16)) * 16
        chunkv = jnp.full((16,), 0, jnp.int32) + chunk
        rowvec = iota + off
        growvec = iota + g * 16
        acc = jnp.zeros((16,), jnp.float32)
        for j in range(EMBED):
            colv = jnp.full((16,), j, jnp.int32)
            cv = plsc.load_gather(crows_v, [growvec, colv])
            av = plsc.load_gather(arows_v, [growvec, colv])
            acc = acc + cv * av
        bc = plsc.load_gather(cb_v, [chunkv, rowvec])
        ba = plsc.load_gather(ab_v, [chunkv, rowvec])
        x = acc + bc + ba
        out_v[g] = 1.0 / (1.0 + jnp.exp(-x))
        return _

    lax.fori_loop(0, _GROUPS, group, None)

    pltpu.sync_copy(out_v, out_hbm.at[pl.ds(wid * _GROUPS, _GROUPS)])


@jax.jit
def _hm_model(crow2d, arow2d, ctab, atab, cbias, abias):
    mesh = plsc.VectorSubcoreMesh(core_axis_name="c", subcore_axis_name="s")
    kfn = functools.partial(
        pl.kernel,
        mesh=mesh,
        compiler_params=pltpu.CompilerParams(
            needs_layout_passes=False, use_tc_tiling_on_sc=False),
        out_type=jax.ShapeDtypeStruct((_NW * _GROUPS, 16), jnp.float32),
        scratch_types=[
            pltpu.VMEM((_NCH, _CH), jnp.int32),
            pltpu.VMEM((_NCH, _CH), jnp.int32),
            pltpu.VMEM((_BPW, EMBED), jnp.float32),
            pltpu.VMEM((_BPW, EMBED), jnp.float32),
            pltpu.VMEM((_NCH, _CH), jnp.float32),
            pltpu.VMEM((_NCH, _CH), jnp.float32),
            pltpu.VMEM((_GROUPS, 16), jnp.float32),
            pltpu.SemaphoreType.DMA,
        ],
    )(_body)
    return kfn(crow2d, arow2d, ctab, atab, cbias, abias)


def kernel(customer_row, article_row, customer_table, article_table,
           customer_bias, article_bias):
    crow2d = customer_row.astype(jnp.int32).reshape(_NW * _NCH, _CH)
    arow2d = article_row.astype(jnp.int32).reshape(_NW * _NCH, _CH)
    cbias = customer_bias.reshape(-1)
    abias = article_bias.reshape(-1)
    out = _hm_model(crow2d, arow2d, customer_table, article_table, cbias, abias)
    return out.reshape(BATCH, 1)
